# baseline (speedup 1.0000x reference)
import jax
import jax.numpy as jnp
from jax.experimental import pallas as pl
from jax.experimental.pallas import tpu as pltpu

QBLK = 512


def _attn_kernel(st_ref, q_ref, k_ref, v_ref, o_ref, sks_ref, stv_ref):
    j = pl.program_id(2)
    d = q_ref.shape[-1]

    @pl.when(j == 0)
    def _pool():
        kb = k_ref[0, 0].astype(jnp.bfloat16)
        vb = v_ref[0, 0].astype(jnp.bfloat16)
        st = st_ref[...]
        sks = jax.lax.dot_general(
            st, kb, (((1,), (0,)), ((), ())),
            preferred_element_type=jnp.float32)
        stv = jax.lax.dot_general(
            st, vb, (((1,), (0,)), ((), ())),
            preferred_element_type=jnp.float32)
        sks_ref[...] = (sks * (1.0 / (d ** 0.5))).astype(jnp.bfloat16)
        stv_ref[...] = stv.astype(jnp.bfloat16)

    q = q_ref[0, 0].astype(jnp.bfloat16)
    s = jax.lax.dot_general(
        q, sks_ref[...], (((1,), (1,)), ((), ())),
        preferred_element_type=jnp.float32)
    e = jnp.exp(s)
    r = jnp.sum(e, axis=-1, keepdims=True)
    o = jax.lax.dot_general(
        e.astype(jnp.bfloat16), stv_ref[...], (((1,), (0,)), ((), ())),
        preferred_element_type=jnp.float32)
    vres = v_ref[0, 0, pl.ds(j * QBLK, QBLK), :]
    o_ref[0, 0] = o / r + vres


def kernel(Q, K, V, mask):
    b, h, n, d = Q.shape
    m2 = 256
    pool = n // m2
    nq = n // QBLK
    st = jnp.where(
        (jnp.arange(n)[None, :] // pool) == jnp.arange(m2)[:, None],
        1.0 / pool, 0.0).astype(jnp.bfloat16)
    out = pl.pallas_call(
        _attn_kernel,
        grid=(b, h, nq),
        in_specs=[
            pl.BlockSpec((m2, n), lambda ib, ih, j: (0, 0)),
            pl.BlockSpec((1, 1, QBLK, d), lambda ib, ih, j: (ib, ih, j, 0)),
            pl.BlockSpec((1, 1, n, d), lambda ib, ih, j: (ib, ih, 0, 0)),
            pl.BlockSpec((1, 1, n, d), lambda ib, ih, j: (ib, ih, 0, 0)),
        ],
        out_specs=pl.BlockSpec((1, 1, QBLK, d), lambda ib, ih, j: (ib, ih, j, 0)),
        out_shape=jax.ShapeDtypeStruct((b, h, n, d), jnp.float32),
        scratch_shapes=[
            pltpu.VMEM((m2, d), jnp.bfloat16),
            pltpu.VMEM((m2, d), jnp.bfloat16),
        ],
        compiler_params=pltpu.CompilerParams(
            dimension_semantics=("arbitrary", "arbitrary", "arbitrary")),
    )(st, Q, K, V)
    return out


# speedup vs baseline: 1.9077x; 1.9077x over previous
import jax
import jax.numpy as jnp
from jax.experimental import pallas as pl
from jax.experimental.pallas import tpu as pltpu

QBLK = 512


def _attn_kernel(st_ref, qt_ref, kt_ref, vt_ref, ot_ref, sks_ref, stvt_ref):
    j = pl.program_id(2)
    d = qt_ref.shape[2]

    @pl.when(j == 0)
    def _pool():
        kb = kt_ref[0, 0].astype(jnp.bfloat16)
        vb = vt_ref[0, 0].astype(jnp.bfloat16)
        st = st_ref[...]
        sks = jax.lax.dot_general(
            st, kb, (((1,), (1,)), ((), ())),
            preferred_element_type=jnp.float32)
        stvt = jax.lax.dot_general(
            vb, st, (((1,), (1,)), ((), ())),
            preferred_element_type=jnp.float32)
        sks_ref[...] = (sks * (1.0 / (d ** 0.5))).astype(jnp.bfloat16)
        stvt_ref[...] = stvt.astype(jnp.bfloat16)

    qt = qt_ref[0, 0].astype(jnp.bfloat16)
    s = jax.lax.dot_general(
        sks_ref[...], qt, (((1,), (0,)), ((), ())),
        preferred_element_type=jnp.float32)
    e = jnp.exp(s)
    r = jnp.sum(e, axis=0, keepdims=True)
    o = jax.lax.dot_general(
        stvt_ref[...], e.astype(jnp.bfloat16), (((1,), (0,)), ((), ())),
        preferred_element_type=jnp.float32)
    vres = vt_ref[0, 0, :, pl.ds(j * QBLK, QBLK)]
    ot_ref[0, 0] = o / r + vres


def kernel(Q, K, V, mask):
    b, h, n, d = Q.shape
    m2 = 256
    pool = n // m2
    nq = n // QBLK
    QT = jnp.swapaxes(Q, 2, 3)
    KT = jnp.swapaxes(K, 2, 3)
    VT = jnp.swapaxes(V, 2, 3)
    st = jnp.where(
        (jnp.arange(n)[None, :] // pool) == jnp.arange(m2)[:, None],
        1.0 / pool, 0.0).astype(jnp.bfloat16)
    out = pl.pallas_call(
        _attn_kernel,
        grid=(b, h, nq),
        in_specs=[
            pl.BlockSpec((m2, n), lambda ib, ih, j: (0, 0)),
            pl.BlockSpec((1, 1, d, QBLK), lambda ib, ih, j: (ib, ih, 0, j)),
            pl.BlockSpec((1, 1, d, n), lambda ib, ih, j: (ib, ih, 0, 0)),
            pl.BlockSpec((1, 1, d, n), lambda ib, ih, j: (ib, ih, 0, 0)),
        ],
        out_specs=pl.BlockSpec(
            (1, 1, d, QBLK), lambda ib, ih, j: (ib, ih, 0, j)),
        out_shape=jax.ShapeDtypeStruct((b, h, d, n), jnp.float32),
        scratch_shapes=[
            pltpu.VMEM((m2, d), jnp.bfloat16),
            pltpu.VMEM((d, m2), jnp.bfloat16),
        ],
        compiler_params=pltpu.CompilerParams(
            dimension_semantics=("arbitrary", "arbitrary", "arbitrary")),
    )(st, QT, KT, VT)
    return jnp.swapaxes(out, 2, 3)


# speedup vs baseline: 2.7472x; 1.4400x over previous
import jax
import jax.numpy as jnp
from jax.experimental import pallas as pl
from jax.experimental.pallas import tpu as pltpu

QBLK = 1024


def _attn_kernel(st_ref, qt_ref, kt_ref, vt_ref, ot_ref, sks_ref, stvt_ref):
    j = pl.program_id(2)
    d = qt_ref.shape[2]

    @pl.when(j == 0)
    def _pool():
        kb = kt_ref[0, 0].astype(jnp.bfloat16)
        vb = vt_ref[0, 0].astype(jnp.bfloat16)
        st = st_ref[...]
        sks = jax.lax.dot_general(
            st, kb, (((1,), (1,)), ((), ())),
            preferred_element_type=jnp.float32)
        stvt = jax.lax.dot_general(
            vb, st, (((1,), (1,)), ((), ())),
            preferred_element_type=jnp.float32)
        sks_ref[...] = (sks * (1.0 / (d ** 0.5))).astype(jnp.bfloat16)
        stvt_ref[...] = stvt.astype(jnp.bfloat16)

    qt = qt_ref[0, 0].astype(jnp.bfloat16)
    s = jax.lax.dot_general(
        sks_ref[...], qt, (((1,), (0,)), ((), ())),
        preferred_element_type=jnp.float32)
    e = jnp.exp(s)
    r = jnp.sum(e, axis=0, keepdims=True)
    o = jax.lax.dot_general(
        stvt_ref[...], e.astype(jnp.bfloat16), (((1,), (0,)), ((), ())),
        preferred_element_type=jnp.float32)
    vres = vt_ref[0, 0, :, pl.ds(j * QBLK, QBLK)]
    ot_ref[0, 0] = o / r + vres


def kernel(Q, K, V, mask):
    b, h, n, d = Q.shape
    m2 = 256
    pool = n // m2
    nq = n // QBLK
    QT = jnp.swapaxes(Q, 2, 3)
    KT = jnp.swapaxes(K, 2, 3)
    VT = jnp.swapaxes(V, 2, 3)
    st = jnp.where(
        (jnp.arange(n)[None, :] // pool) == jnp.arange(m2)[:, None],
        1.0 / pool, 0.0).astype(jnp.bfloat16)
    out = pl.pallas_call(
        _attn_kernel,
        grid=(b, h, nq),
        in_specs=[
            pl.BlockSpec((m2, n), lambda ib, ih, j: (0, 0)),
            pl.BlockSpec((1, 1, d, QBLK), lambda ib, ih, j: (ib, ih, 0, j)),
            pl.BlockSpec((1, 1, d, n), lambda ib, ih, j: (ib, ih, 0, 0)),
            pl.BlockSpec((1, 1, d, n), lambda ib, ih, j: (ib, ih, 0, 0)),
        ],
        out_specs=pl.BlockSpec(
            (1, 1, d, QBLK), lambda ib, ih, j: (ib, ih, 0, j)),
        out_shape=jax.ShapeDtypeStruct((b, h, d, n), jnp.float32),
        scratch_shapes=[
            pltpu.VMEM((m2, d), jnp.bfloat16),
            pltpu.VMEM((d, m2), jnp.bfloat16),
        ],
        compiler_params=pltpu.CompilerParams(
            dimension_semantics=("arbitrary", "arbitrary", "arbitrary")),
    )(st, QT, KT, VT)
    return jnp.swapaxes(out, 2, 3)


# speedup vs baseline: 3.5363x; 1.2872x over previous
import jax
import jax.numpy as jnp
from jax.experimental import pallas as pl
from jax.experimental.pallas import tpu as pltpu

QBLK = 2048


def _attn_kernel(st_ref, qt_ref, kt_ref, vt_ref, ot_ref, sks_ref, stvt_ref):
    j = pl.program_id(2)
    d = qt_ref.shape[2]

    @pl.when(j == 0)
    def _pool():
        kb = kt_ref[0, 0].astype(jnp.bfloat16)
        vb = vt_ref[0, 0].astype(jnp.bfloat16)
        st = st_ref[...]
        sks = jax.lax.dot_general(
            st, kb, (((1,), (1,)), ((), ())),
            preferred_element_type=jnp.float32)
        stvt = jax.lax.dot_general(
            vb, st, (((1,), (1,)), ((), ())),
            preferred_element_type=jnp.float32)
        sks_ref[...] = (sks * (1.0 / (d ** 0.5))).astype(jnp.bfloat16)
        stvt_ref[...] = stvt.astype(jnp.bfloat16)

    qt = qt_ref[0, 0].astype(jnp.bfloat16)
    s = jax.lax.dot_general(
        sks_ref[...], qt, (((1,), (0,)), ((), ())),
        preferred_element_type=jnp.float32)
    e = jnp.exp(s)
    r = jnp.sum(e, axis=0, keepdims=True)
    o = jax.lax.dot_general(
        stvt_ref[...], e.astype(jnp.bfloat16), (((1,), (0,)), ((), ())),
        preferred_element_type=jnp.float32)
    vres = vt_ref[0, 0, :, pl.ds(j * QBLK, QBLK)]
    ot_ref[0, 0] = o / r + vres


def kernel(Q, K, V, mask):
    b, h, n, d = Q.shape
    m2 = 256
    pool = n // m2
    nq = n // QBLK
    QT = jnp.swapaxes(Q, 2, 3)
    KT = jnp.swapaxes(K, 2, 3)
    VT = jnp.swapaxes(V, 2, 3)
    st = jnp.where(
        (jnp.arange(n)[None, :] // pool) == jnp.arange(m2)[:, None],
        1.0 / pool, 0.0).astype(jnp.bfloat16)
    out = pl.pallas_call(
        _attn_kernel,
        grid=(b, h, nq),
        in_specs=[
            pl.BlockSpec((m2, n), lambda ib, ih, j: (0, 0)),
            pl.BlockSpec((1, 1, d, QBLK), lambda ib, ih, j: (ib, ih, 0, j)),
            pl.BlockSpec((1, 1, d, n), lambda ib, ih, j: (ib, ih, 0, 0)),
            pl.BlockSpec((1, 1, d, n), lambda ib, ih, j: (ib, ih, 0, 0)),
        ],
        out_specs=pl.BlockSpec(
            (1, 1, d, QBLK), lambda ib, ih, j: (ib, ih, 0, j)),
        out_shape=jax.ShapeDtypeStruct((b, h, d, n), jnp.float32),
        scratch_shapes=[
            pltpu.VMEM((m2, d), jnp.bfloat16),
            pltpu.VMEM((d, m2), jnp.bfloat16),
        ],
        compiler_params=pltpu.CompilerParams(
            dimension_semantics=("arbitrary", "arbitrary", "arbitrary")),
    )(st, QT, KT, VT)
    return jnp.swapaxes(out, 2, 3)


# speedup vs baseline: 5.9828x; 1.6919x over previous
import jax
import jax.numpy as jnp
from jax.experimental import pallas as pl
from jax.experimental.pallas import tpu as pltpu

QBLK = 4096


def _attn_kernel(st_ref, qt_ref, kt_ref, vt_ref, ot_ref, sks_ref, stvt_ref):
    j = pl.program_id(2)
    d = qt_ref.shape[2]

    @pl.when(j == 0)
    def _pool():
        kb = kt_ref[0, 0].astype(jnp.bfloat16)
        vb = vt_ref[0, 0].astype(jnp.bfloat16)
        st = st_ref[...]
        sks = jax.lax.dot_general(
            st, kb, (((1,), (1,)), ((), ())),
            preferred_element_type=jnp.float32)
        stvt = jax.lax.dot_general(
            vb, st, (((1,), (1,)), ((), ())),
            preferred_element_type=jnp.float32)
        sks_ref[...] = (sks * (1.0 / (d ** 0.5))).astype(jnp.bfloat16)
        stvt_ref[...] = stvt.astype(jnp.bfloat16)

    qt = qt_ref[0, 0].astype(jnp.bfloat16)
    s = jax.lax.dot_general(
        sks_ref[...], qt, (((1,), (0,)), ((), ())),
        preferred_element_type=jnp.float32)
    e = jnp.exp(s)
    r = jnp.sum(e, axis=0, keepdims=True)
    o = jax.lax.dot_general(
        stvt_ref[...], e.astype(jnp.bfloat16), (((1,), (0,)), ((), ())),
        preferred_element_type=jnp.float32)
    vres = vt_ref[0, 0, :, pl.ds(j * QBLK, QBLK)]
    ot_ref[0, 0] = o / r + vres


def kernel(Q, K, V, mask):
    b, h, n, d = Q.shape
    m2 = 256
    pool = n // m2
    nq = n // QBLK
    QT = jnp.swapaxes(Q, 2, 3)
    KT = jnp.swapaxes(K, 2, 3)
    VT = jnp.swapaxes(V, 2, 3)
    st = jnp.where(
        (jnp.arange(n)[None, :] // pool) == jnp.arange(m2)[:, None],
        1.0 / pool, 0.0).astype(jnp.bfloat16)
    out = pl.pallas_call(
        _attn_kernel,
        grid=(b, h, nq),
        in_specs=[
            pl.BlockSpec((m2, n), lambda ib, ih, j: (0, 0)),
            pl.BlockSpec((1, 1, d, QBLK), lambda ib, ih, j: (ib, ih, 0, j)),
            pl.BlockSpec((1, 1, d, n), lambda ib, ih, j: (ib, ih, 0, 0)),
            pl.BlockSpec((1, 1, d, n), lambda ib, ih, j: (ib, ih, 0, 0)),
        ],
        out_specs=pl.BlockSpec(
            (1, 1, d, QBLK), lambda ib, ih, j: (ib, ih, 0, j)),
        out_shape=jax.ShapeDtypeStruct((b, h, d, n), jnp.float32),
        scratch_shapes=[
            pltpu.VMEM((m2, d), jnp.bfloat16),
            pltpu.VMEM((d, m2), jnp.bfloat16),
        ],
        compiler_params=pltpu.CompilerParams(
            dimension_semantics=("arbitrary", "arbitrary", "arbitrary")),
    )(st, QT, KT, VT)
    return jnp.swapaxes(out, 2, 3)
